# Initial kernel scaffold; baseline (speedup 1.0000x reference)
import jax
import jax.numpy as jnp
from jax.experimental import pallas as pl


def kernel(input_ids, word_embeddings):
    raise NotImplementedError("write your pallas kernel here")


# speedup vs baseline: 2.9654x; 2.9654x over previous
import functools

import jax
import jax.numpy as jnp
from jax import lax
from jax.experimental import pallas as pl
from jax.experimental.pallas import tpu as pltpu
from jax.experimental.pallas import tpu_sc as plsc

D = 128
NC, NS = 2, 16
NW = NC * NS
CH = 128


@functools.partial(jax.jit, static_argnames=("chunks",))
def _lookup(idx2d, table, *, chunks):
    n_tokens = NW * chunks * CH
    mesh = plsc.VectorSubcoreMesh(core_axis_name="c", subcore_axis_name="s")

    @functools.partial(
        pl.kernel,
        out_type=jax.ShapeDtypeStruct((n_tokens, D), jnp.float32),
        mesh=mesh,
        scratch_types=[
            pltpu.VMEM((chunks, CH), jnp.int32),
            pltpu.VMEM((CH, D), jnp.float32),
            pltpu.SemaphoreType.DMA,
        ],
    )
    def body(table_hbm, idx_hbm, out_hbm, idx_v, rows_v, sem):
        wid = lax.axis_index("s") * NC + lax.axis_index("c")
        pltpu.sync_copy(idx_hbm.at[wid], idx_v)
        base = wid * chunks * CH

        @pl.loop(0, chunks)
        def _(c):
            pltpu.async_copy(table_hbm.at[idx_v.at[c]], rows_v, sem).wait()
            pltpu.sync_copy(rows_v, out_hbm.at[pl.ds(base + c * CH, CH)])

    return body(table, idx2d)


def kernel(input_ids, word_embeddings):
    batch, hist = input_ids.shape
    n_tokens = batch * hist
    chunks = n_tokens // (NW * CH)
    idx2d = input_ids.reshape(NW, chunks, CH).astype(jnp.int32)
    out = _lookup(idx2d, word_embeddings, chunks=chunks)
    return out.reshape(batch, hist, D)


# speedup vs baseline: 3.3108x; 1.1165x over previous
import functools

import jax
import jax.numpy as jnp
from jax import lax
from jax.experimental import pallas as pl
from jax.experimental.pallas import tpu as pltpu
from jax.experimental.pallas import tpu_sc as plsc

D = 128
NC, NS = 2, 16
NW = NC * NS
CH = 128
NBUF = 5


@functools.partial(jax.jit, static_argnames=("chunks",))
def _lookup(idx2d, table, *, chunks):
    n_tokens = NW * chunks * CH
    mesh = plsc.VectorSubcoreMesh(core_axis_name="c", subcore_axis_name="s")

    @functools.partial(
        pl.kernel,
        out_type=jax.ShapeDtypeStruct((n_tokens, D), jnp.float32),
        mesh=mesh,
        scratch_types=[
            pltpu.VMEM((chunks, CH), jnp.int32),
            pltpu.VMEM((NBUF, CH, D), jnp.float32),
            pltpu.SemaphoreType.DMA((NBUF,)),
            pltpu.SemaphoreType.DMA((NBUF,)),
        ],
    )
    def body(table_hbm, idx_hbm, out_hbm, idx_v, rows_v, gsem, wsem):
        wid = lax.axis_index("s") * NC + lax.axis_index("c")
        pltpu.sync_copy(idx_hbm.at[wid], idx_v)
        base = wid * chunks * CH

        def fire_gather(c, b):
            pltpu.async_copy(table_hbm.at[idx_v.at[c]], rows_v.at[b],
                             gsem.at[b])

        def wait_gather(b):
            pltpu.make_async_copy(table_hbm.at[idx_v.at[0]], rows_v.at[b],
                                  gsem.at[b]).wait()

        def fire_write(c, b):
            pltpu.async_copy(rows_v.at[b], out_hbm.at[pl.ds(base + c * CH, CH)],
                             wsem.at[b])

        def wait_write(b):
            pltpu.make_async_copy(rows_v.at[b], out_hbm.at[pl.ds(base, CH)],
                                  wsem.at[b]).wait()

        for b in range(NBUF):
            fire_gather(b, b)

        @pl.loop(0, chunks - NBUF, step=NBUF)
        def _(c0):
            for b in range(NBUF):
                wait_gather(b)
                fire_write(c0 + b, b)
            for b in range(NBUF):
                wait_write(b)
                fire_gather(c0 + NBUF + b, b)

        for b in range(NBUF):
            wait_gather(b)
            fire_write(chunks - NBUF + b, b)
        for b in range(NBUF):
            wait_write(b)

    return body(table, idx2d)


def kernel(input_ids, word_embeddings):
    batch, hist = input_ids.shape
    n_tokens = batch * hist
    chunks = n_tokens // (NW * CH)
    idx2d = input_ids.reshape(NW, chunks, CH).astype(jnp.int32)
    out = _lookup(idx2d, word_embeddings, chunks=chunks)
    return out.reshape(batch, hist, D)
